# Initial kernel scaffold; baseline (speedup 1.0000x reference)
import functools

import jax
import jax.numpy as jnp
from jax import lax
from jax.experimental import pallas as pl
from jax.experimental.pallas import tpu as pltpu
from jax.experimental.pallas import tpu_sc as plsc

LANES = 16
GATHER_SUB = 128


def _build_table(emb_obs, emb_feat, b_val):
    max_n, d = emb_obs.shape
    di = emb_feat.shape[0]

    def body(eo_ref, ef_ref, b_ref, out_ref):
        eo = eo_ref[...]
        ef = ef_ref[...]
        b = b_ref[...]
        tbl = eo[:, None, :] + ef[None, :, :] + b[None, None, :]
        out_ref[...] = tbl.reshape(max_n * di, d)

    return pl.pallas_call(
        body,
        out_shape=jax.ShapeDtypeStruct((max_n * di, d), jnp.float32),
    )(emb_obs, emb_feat, b_val)


@functools.partial(jax.jit, static_argnames=("di",))
def _sc_embed(tbl, obs, feat, valf, w, *, di):
    n, = obs.shape
    _, d = tbl.shape
    info = plsc.get_sparse_core_info()
    nw = info.num_cores * info.num_subcores
    chunk = 512
    assert n % (nw * chunk) == 0
    nchunks = n // (nw * chunk)
    n_sub = chunk // GATHER_SUB
    n_vec = chunk // LANES
    mesh = plsc.VectorSubcoreMesh(core_axis_name="c", subcore_axis_name="s")

    @functools.partial(
        pl.kernel,
        out_type=jax.ShapeDtypeStruct((n, d), jnp.float32),
        mesh=mesh,
        scratch_types=[
            pltpu.VMEM((chunk,), jnp.int32),
            pltpu.VMEM((chunk,), jnp.int32),
            pltpu.VMEM((chunk,), jnp.int32),
            pltpu.VMEM((chunk,), jnp.float32),
            pltpu.VMEM((d,), jnp.float32),
            pltpu.VMEM((chunk, d), jnp.float32),
            pltpu.SemaphoreType.DMA,
        ],
    )
    def k(tbl_hbm, obs_hbm, feat_hbm, val_hbm, w_hbm, out_hbm,
          obs_v, feat_v, idx_v, val_v, w_v, rows_v, sem):
        wid = lax.axis_index("s") * info.num_cores + lax.axis_index("c")
        pltpu.sync_copy(w_hbm, w_v)

        def chunk_body(g, _):
            base = (wid * nchunks + g) * chunk
            pltpu.sync_copy(obs_hbm.at[pl.ds(base, chunk)], obs_v)
            pltpu.sync_copy(feat_hbm.at[pl.ds(base, chunk)], feat_v)
            pltpu.sync_copy(val_hbm.at[pl.ds(base, chunk)], val_v)
            for i in range(n_vec):
                sl = pl.ds(i * LANES, LANES)
                idx_v[sl] = obs_v[sl] * di + feat_v[sl]
            copies = []
            for j in range(n_sub):
                sl = pl.ds(j * GATHER_SUB, GATHER_SUB)
                copies.append(
                    pltpu.async_copy(tbl_hbm.at[idx_v.at[sl]], rows_v.at[sl], sem))
            for c in copies:
                c.wait()

            wregs = [w_v[pl.ds(t * LANES, LANES)] for t in range(d // LANES)]

            def row_body(r, _):
                vv = plsc.load_gather(val_v, [jnp.full((LANES,), r, jnp.int32)])
                for t in range(d // LANES):
                    sl = pl.ds(t * LANES, LANES)
                    rows_v[r, sl] = rows_v[r, sl] + vv * wregs[t]
                return 0

            lax.fori_loop(0, chunk, row_body, 0)
            pltpu.sync_copy(rows_v, out_hbm.at[pl.ds(base, chunk)])
            return 0

        lax.fori_loop(0, nchunks, chunk_body, 0)

    return k(tbl, obs, feat, valf, w)


def kernel(val, obs_idx, feat_idx, W_val, b_val, emb_obs, emb_feat):
    b, l, _ = val.shape
    n = b * l
    d = emb_obs.shape[1]
    di = emb_feat.shape[0]
    tbl = _build_table(emb_obs, emb_feat, b_val)
    out = _sc_embed(
        tbl,
        obs_idx.reshape(n).astype(jnp.int32),
        feat_idx.reshape(n).astype(jnp.int32),
        val.reshape(n),
        W_val.reshape(d),
        di=di,
    )
    return out.reshape(b, l, d)


# speedup vs baseline: 6.5187x; 6.5187x over previous
import functools

import jax
import jax.numpy as jnp
from jax import lax
from jax.experimental import pallas as pl
from jax.experimental.pallas import tpu as pltpu
from jax.experimental.pallas import tpu_sc as plsc

LANES = 16
GATHER_SUB = 128


def _build_table(emb_obs, emb_feat, b_val):
    max_n, d = emb_obs.shape
    di = emb_feat.shape[0]

    def body(eo_ref, ef_ref, b_ref, out_ref):
        eo = eo_ref[...]
        ef = ef_ref[...]
        b = b_ref[...]
        tbl = eo[:, None, :] + ef[None, :, :] + b[None, None, :]
        out_ref[...] = tbl.reshape(max_n * di, d)

    return pl.pallas_call(
        body,
        out_shape=jax.ShapeDtypeStruct((max_n * di, d), jnp.float32),
    )(emb_obs, emb_feat, b_val)


@functools.partial(jax.jit, static_argnames=("di",))
def _sc_embed(tbl, obs, feat, valf, w, *, di):
    n, = obs.shape
    _, d = tbl.shape
    info = plsc.get_sparse_core_info()
    nw = info.num_cores * info.num_subcores
    chunk = 512
    assert n % (nw * chunk) == 0
    nchunks = n // (nw * chunk)
    n_sub = chunk // GATHER_SUB
    n_vec = chunk // LANES
    mesh = plsc.VectorSubcoreMesh(core_axis_name="c", subcore_axis_name="s")

    @functools.partial(
        pl.kernel,
        out_type=jax.ShapeDtypeStruct((n, d), jnp.float32),
        mesh=mesh,
        compiler_params=pltpu.CompilerParams(use_tc_tiling_on_sc=False),
        scratch_types=[
            pltpu.VMEM((chunk,), jnp.int32),
            pltpu.VMEM((chunk,), jnp.int32),
            pltpu.VMEM((chunk,), jnp.int32),
            pltpu.VMEM((chunk,), jnp.float32),
            pltpu.VMEM((d,), jnp.float32),
            pltpu.VMEM((chunk, d), jnp.float32),
            pltpu.SemaphoreType.DMA,
        ],
    )
    def k(tbl_hbm, obs_hbm, feat_hbm, val_hbm, w_hbm, out_hbm,
          obs_v, feat_v, idx_v, val_v, w_v, rows_v, sem):
        wid = lax.axis_index("s") * info.num_cores + lax.axis_index("c")
        pltpu.sync_copy(w_hbm, w_v)

        def chunk_body(g, _):
            base = (wid * nchunks + g) * chunk
            pltpu.sync_copy(obs_hbm.at[pl.ds(base, chunk)], obs_v)
            pltpu.sync_copy(feat_hbm.at[pl.ds(base, chunk)], feat_v)
            pltpu.sync_copy(val_hbm.at[pl.ds(base, chunk)], val_v)
            for i in range(n_vec):
                sl = pl.ds(i * LANES, LANES)
                idx_v[sl] = obs_v[sl] * di + feat_v[sl]
            copies = []
            for j in range(n_sub):
                sl = pl.ds(j * GATHER_SUB, GATHER_SUB)
                copies.append(
                    pltpu.async_copy(tbl_hbm.at[idx_v.at[sl]], rows_v.at[sl], sem))
            for c in copies:
                c.wait()

            wregs = [w_v[pl.ds(t * LANES, LANES)] for t in range(d // LANES)]

            def group_body(i, _):
                v16 = val_v[pl.ds(i * LANES, LANES)]
                for rr in range(LANES):
                    vv = lax.gather(
                        v16, jnp.full((LANES, 1), rr, jnp.int32),
                        dimension_numbers=lax.GatherDimensionNumbers(
                            offset_dims=(), collapsed_slice_dims=(0,),
                            start_index_map=(0,)),
                        slice_sizes=(1,),
                        mode=lax.GatherScatterMode.PROMISE_IN_BOUNDS)
                    r = i * LANES + rr
                    for t in range(d // LANES):
                        sl = pl.ds(t * LANES, LANES)
                        rows_v[r, sl] = rows_v[r, sl] + vv * wregs[t]
                return 0

            lax.fori_loop(0, n_vec, group_body, 0)
            pltpu.sync_copy(rows_v, out_hbm.at[pl.ds(base, chunk)])
            return 0

        lax.fori_loop(0, nchunks, chunk_body, 0)

    return k(tbl, obs, feat, valf, w)


def kernel(val, obs_idx, feat_idx, W_val, b_val, emb_obs, emb_feat):
    b, l, _ = val.shape
    n = b * l
    d = emb_obs.shape[1]
    di = emb_feat.shape[0]
    tbl = _build_table(emb_obs, emb_feat, b_val)
    out = _sc_embed(
        tbl,
        obs_idx.reshape(n).astype(jnp.int32),
        feat_idx.reshape(n).astype(jnp.int32),
        val.reshape(n),
        W_val.reshape(d),
        di=di,
    )
    return out.reshape(b, l, d)


# speedup vs baseline: 8.1075x; 1.2437x over previous
import functools

import jax
import jax.numpy as jnp
from jax import lax
from jax.experimental import pallas as pl
from jax.experimental.pallas import tpu as pltpu
from jax.experimental.pallas import tpu_sc as plsc

LANES = 16
GATHER_SUB = 128


def _build_table(emb_obs, emb_feat, b_val):
    max_n, d = emb_obs.shape
    di = emb_feat.shape[0]

    def body(eo_ref, ef_ref, b_ref, out_ref):
        tbl = eo_ref[...][:, None, :] + ef_ref[...][None, :, :] + b_ref[...][None, None, :]
        out_ref[...] = tbl.reshape(max_n * di, d)

    return pl.pallas_call(
        body,
        out_shape=jax.ShapeDtypeStruct((max_n * di, d), jnp.float32),
    )(emb_obs, emb_feat, b_val)


@functools.partial(jax.jit, static_argnames=("di",))
def _sc_embed(tbl, obs, feat, valf, w, *, di):
    n, = obs.shape
    _, d = tbl.shape
    info = plsc.get_sparse_core_info()
    nw = info.num_cores * info.num_subcores
    chunk = 512
    assert n % (nw * chunk) == 0
    nchunks = n // (nw * chunk)
    assert nchunks % 2 == 0 and nchunks >= 4
    n_sub = chunk // GATHER_SUB
    n_vec = chunk // LANES
    nt = d // LANES
    mesh = plsc.VectorSubcoreMesh(core_axis_name="c", subcore_axis_name="s")

    @functools.partial(
        pl.kernel,
        out_type=jax.ShapeDtypeStruct((n, d), jnp.float32),
        mesh=mesh,
        compiler_params=pltpu.CompilerParams(use_tc_tiling_on_sc=False),
        scratch_types=[
            pltpu.VMEM((chunk,), jnp.int32),
            pltpu.VMEM((chunk,), jnp.int32),
            pltpu.VMEM((chunk,), jnp.int32),
            pltpu.VMEM((chunk,), jnp.int32),
            pltpu.VMEM((chunk,), jnp.int32),
            pltpu.VMEM((chunk,), jnp.int32),
            pltpu.VMEM((chunk,), jnp.float32),
            pltpu.VMEM((chunk,), jnp.float32),
            pltpu.VMEM((d,), jnp.float32),
            pltpu.VMEM((chunk, d), jnp.float32),
            pltpu.VMEM((chunk, d), jnp.float32),
            pltpu.SemaphoreType.DMA,
            pltpu.SemaphoreType.DMA,
            pltpu.SemaphoreType.DMA,
            pltpu.SemaphoreType.DMA,
            pltpu.SemaphoreType.DMA,
            pltpu.SemaphoreType.DMA,
        ],
    )
    def k(tbl_hbm, obs_hbm, feat_hbm, val_hbm, w_hbm, out_hbm,
          obs0, obs1, feat0, feat1, idx0, idx1, val0, val1, w_v,
          rows0, rows1, sof0, sof1, sv0, sv1, sg0, sg1):
        OBS, FEAT, IDX, VAL = [obs0, obs1], [feat0, feat1], [idx0, idx1], [val0, val1]
        ROWS, SOF, SV, SG = [rows0, rows1], [sof0, sof1], [sv0, sv1], [sg0, sg1]
        wid = lax.axis_index("s") * info.num_cores + lax.axis_index("c")
        pltpu.sync_copy(w_hbm, w_v)
        wregs = [w_v[pl.ds(t * LANES, LANES)] for t in range(nt)]

        def row_base(g):
            return (wid * nchunks + g) * chunk

        def issue_of(g, p):
            base = row_base(g)
            pltpu.async_copy(obs_hbm.at[pl.ds(base, chunk)], OBS[p], SOF[p])
            pltpu.async_copy(feat_hbm.at[pl.ds(base, chunk)], FEAT[p], SOF[p])

        def wait_of(p):
            pltpu.make_async_copy(obs_hbm.at[pl.ds(0, chunk)], OBS[p], SOF[p]).wait()
            pltpu.make_async_copy(feat_hbm.at[pl.ds(0, chunk)], FEAT[p], SOF[p]).wait()

        def issue_v(g, p):
            pltpu.async_copy(val_hbm.at[pl.ds(row_base(g), chunk)], VAL[p], SV[p])

        def wait_v(p):
            pltpu.make_async_copy(val_hbm.at[pl.ds(0, chunk)], VAL[p], SV[p]).wait()

        def combine(p):
            for i in range(n_vec):
                sl = pl.ds(i * LANES, LANES)
                IDX[p][sl] = OBS[p][sl] * di + FEAT[p][sl]

        def issue_gather(p):
            for j in range(n_sub):
                sl = pl.ds(j * GATHER_SUB, GATHER_SUB)
                pltpu.async_copy(tbl_hbm.at[IDX[p].at[sl]], ROWS[p].at[sl], SG[p])

        def wait_gather(p):
            for j in range(n_sub):
                sl = pl.ds(j * GATHER_SUB, GATHER_SUB)
                pltpu.make_async_copy(tbl_hbm.at[IDX[p].at[sl]], ROWS[p].at[sl], SG[p]).wait()

        def compute(p):
            wait_v(p)
            rows_v = ROWS[p]

            def group_body(i, _):
                v16 = VAL[p][pl.ds(i * LANES, LANES)]
                for rr in range(LANES):
                    vv = lax.gather(
                        v16, jnp.full((LANES, 1), rr, jnp.int32),
                        dimension_numbers=lax.GatherDimensionNumbers(
                            offset_dims=(), collapsed_slice_dims=(0,),
                            start_index_map=(0,)),
                        slice_sizes=(1,),
                        mode=lax.GatherScatterMode.PROMISE_IN_BOUNDS)
                    r = i * LANES + rr
                    for t in range(nt):
                        sl = pl.ds(t * LANES, LANES)
                        rows_v[r, sl] = rows_v[r, sl] + vv * wregs[t]
                return 0

            lax.fori_loop(0, n_vec, group_body, 0)

        def out_sync(g, p):
            pltpu.sync_copy(ROWS[p], out_hbm.at[pl.ds(row_base(g), chunk)])

        issue_of(0, 0)
        issue_v(0, 0)
        issue_of(1, 1)
        issue_v(1, 1)
        wait_of(0)
        combine(0)
        issue_gather(0)

        def pair_body(g2, _):
            g0 = 2 * g2
            wait_of(1)
            combine(1)
            issue_gather(1)
            wait_gather(0)
            issue_of(g0 + 2, 0)
            compute(0)
            out_sync(g0, 0)
            issue_v(g0 + 2, 0)
            wait_of(0)
            combine(0)
            issue_gather(0)
            wait_gather(1)
            issue_of(g0 + 3, 1)
            compute(1)
            out_sync(g0 + 1, 1)
            issue_v(g0 + 3, 1)
            return 0

        lax.fori_loop(0, nchunks // 2 - 1, pair_body, 0)

        wait_of(1)
        combine(1)
        issue_gather(1)
        wait_gather(0)
        compute(0)
        out_sync(nchunks - 2, 0)
        wait_gather(1)
        compute(1)
        out_sync(nchunks - 1, 1)

    return k(tbl, obs, feat, valf, w)


def kernel(val, obs_idx, feat_idx, W_val, b_val, emb_obs, emb_feat):
    b, l, _ = val.shape
    n = b * l
    d = emb_obs.shape[1]
    di = emb_feat.shape[0]
    tbl = _build_table(emb_obs, emb_feat, b_val)
    out = _sc_embed(
        tbl,
        obs_idx.reshape(n).astype(jnp.int32),
        feat_idx.reshape(n).astype(jnp.int32),
        val.reshape(n),
        W_val.reshape(d),
        di=di,
    )
    return out.reshape(b, l, d)
